# baseline (device time: 241562 ns/iter reference)
import jax
import jax.numpy as jnp
from jax import lax
from jax.experimental import pallas as pl
from jax.experimental.pallas import tpu as pltpu

N = 16
C = 192
D = 512
H = 1024
E_LOC = 8
PAY = 520
T = 2048

BF = jnp.bfloat16


def _expert_chunk(chunk, eW_ref):
    xp = chunk[:, :D]
    acc = None
    for j in range(E_LOC):
        contrib = jnp.dot(xp * chunk[:, D + j][:, None], eW_ref[j],
                          preferred_element_type=jnp.float32)
        acc = contrib if acc is None else acc + contrib
    return acc.astype(BF)


def _pack_chunk(k, flat, val):
    rows = lax.broadcasted_iota(jnp.int32, (C, T), 0) + k * C
    p_mat = (rows == flat).astype(BF)
    return jnp.dot(p_mat, val, preferred_element_type=jnp.float32).astype(BF)


def _unsort_add(j, flat, chunk, out_ref):
    hh = T // 2
    for half in range(2):
        fl = flat[half * hh:(half + 1) * hh][:, None]
        cols = lax.broadcasted_iota(jnp.int32, (hh, C), 1) + j * C
        g = (cols == fl).astype(BF)
        prod = jnp.dot(g, chunk, preferred_element_type=jnp.float32)
        sl = pl.ds(half * hh, hh)
        out_ref[sl, :] = out_ref[sl, :] + prod


def _body(val_ref, flat_ref, x_ref, shW_ref, eW_ref, out_ref,
          xsend_ref, recv_ref, yrecv_ref, ybuf_ref,
          ss1, rs1, ss2, rs2):
    me = lax.axis_index("i")

    barrier = pltpu.get_barrier_semaphore()
    for k in range(1, N):
        nbr = (me + k) % N
        pl.semaphore_signal(barrier, inc=1, device_id=(nbr,),
                            device_id_type=pl.DeviceIdType.MESH)
    pl.semaphore_wait(barrier, N - 1)

    val = val_ref[...]
    flat = flat_ref[0, :]

    send1 = []
    for k in range(1, N):
        dst = (me + k) % N
        xsend_ref[pl.ds(k * C, C), :] = _pack_chunk(k, flat, val)
        r = pltpu.make_async_remote_copy(
            src_ref=xsend_ref.at[pl.ds(k * C, C), :],
            dst_ref=recv_ref.at[pl.ds((N - k) * C, C), :],
            send_sem=ss1.at[k],
            recv_sem=rs1.at[N - k],
            device_id=(dst,),
            device_id_type=pl.DeviceIdType.MESH,
        )
        r.start()
        send1.append(r)

    out_ref[...] = jnp.dot(x_ref[...], shW_ref[...],
                           preferred_element_type=jnp.float32)

    xsend_ref[pl.ds(0, C), :] = _pack_chunk(0, flat, val)
    ybuf_ref[0] = _expert_chunk(xsend_ref[pl.ds(0, C), :], eW_ref)
    _unsort_add(0, flat, ybuf_ref[0], out_ref)

    send2 = []
    for j in range(1, N):
        s = (me + j) % N
        rwait = pltpu.make_async_remote_copy(
            src_ref=xsend_ref.at[pl.ds(0, C), :],
            dst_ref=recv_ref.at[pl.ds(j * C, C), :],
            send_sem=ss1.at[j],
            recv_sem=rs1.at[j],
            device_id=(s,),
            device_id_type=pl.DeviceIdType.MESH,
        )
        rwait.wait_recv()

        slot = (j - 1) % 2
        if len(send2) >= 2:
            send2[-2].wait_send()
        ybuf_ref[slot] = _expert_chunk(recv_ref[pl.ds(j * C, C), :], eW_ref)

        r = pltpu.make_async_remote_copy(
            src_ref=ybuf_ref.at[slot],
            dst_ref=yrecv_ref.at[pl.ds((N - j) * C, C), :],
            send_sem=ss2.at[j],
            recv_sem=rs2.at[N - j],
            device_id=(s,),
            device_id_type=pl.DeviceIdType.MESH,
        )
        r.start()
        send2.append(r)

        jj = N - j
        rwait = pltpu.make_async_remote_copy(
            src_ref=ybuf_ref.at[0],
            dst_ref=yrecv_ref.at[pl.ds(jj * C, C), :],
            send_sem=ss2.at[jj],
            recv_sem=rs2.at[jj],
            device_id=((me + jj) % N,),
            device_id_type=pl.DeviceIdType.MESH,
        )
        rwait.wait_recv()
        _unsort_add(jj, flat, yrecv_ref[pl.ds(jj * C, C), :], out_ref)

    for r in send1:
        r.wait_send()
    for r in send2[-2:]:
        r.wait_send()


def kernel(x, router_W, route_idx, expert_W, shared_W):
    me = lax.axis_index("i")

    scores = jnp.dot(x, router_W, preferred_element_type=jnp.float32)
    scores = scores - scores.max(axis=1, keepdims=True)
    probs = jnp.exp(scores)
    probs = probs / probs.sum(axis=1, keepdims=True)

    e = route_idx[:, 0]
    n_exp = router_W.shape[1]
    oh_full = (e[:, None] == jnp.arange(n_exp, dtype=e.dtype)[None, :])
    p = jnp.sum(probs * oh_full.astype(probs.dtype), axis=1)

    dst = e // E_LOC
    eloc = e % E_LOC
    rel = (dst - me) % N

    oh_dst = (dst[:, None] == jnp.arange(N, dtype=dst.dtype)[None, :]).astype(jnp.int32)
    j_slot = jnp.sum(jnp.cumsum(oh_dst, axis=0) * oh_dst, axis=1) - 1
    flat = jnp.where(j_slot < C, rel * C + j_slot, N * C)

    xp = (x * p[:, None]).astype(BF)
    oh_e = (eloc[:, None] == jnp.arange(E_LOC, dtype=eloc.dtype)[None, :]).astype(BF)
    val = jnp.concatenate([xp, oh_e], axis=1)
    flat_i = flat.astype(jnp.int32)[None, :]

    out = pl.pallas_call(
        _body,
        out_shape=jax.ShapeDtypeStruct((T, H), jnp.float32),
        in_specs=[
            pl.BlockSpec(memory_space=pltpu.VMEM),
            pl.BlockSpec(memory_space=pltpu.VMEM),
            pl.BlockSpec(memory_space=pltpu.VMEM),
            pl.BlockSpec(memory_space=pltpu.VMEM),
            pl.BlockSpec(memory_space=pltpu.VMEM),
        ],
        out_specs=pl.BlockSpec(memory_space=pltpu.VMEM),
        scratch_shapes=[
            pltpu.VMEM((N * C, PAY), BF),
            pltpu.VMEM((N * C, PAY), BF),
            pltpu.VMEM((N * C, H), BF),
            pltpu.VMEM((2, C, H), BF),
            pltpu.SemaphoreType.DMA((N,)),
            pltpu.SemaphoreType.DMA((N,)),
            pltpu.SemaphoreType.DMA((N,)),
            pltpu.SemaphoreType.DMA((N,)),
        ],
        compiler_params=pltpu.CompilerParams(
            collective_id=0, vmem_limit_bytes=36 * 1024 * 1024),
    )(val, flat_i, x.astype(BF), shared_W.astype(BF), expert_W.astype(BF))

    return out


# device time: 186279 ns/iter; 1.2968x vs baseline; 1.2968x over previous
import jax
import jax.numpy as jnp
from jax import lax
from jax.experimental import pallas as pl
from jax.experimental.pallas import tpu as pltpu

N = 16
C = 192
D = 512
H = 1024
E_LOC = 8
PAY = 520
T = 2048

BF = jnp.bfloat16


def _expert_chunk(chunk, eW_ref):
    xp = chunk[:, :D]
    acc = None
    for j in range(E_LOC):
        contrib = jnp.dot(xp * chunk[:, D + j][:, None], eW_ref[j],
                          preferred_element_type=jnp.float32)
        acc = contrib if acc is None else acc + contrib
    return acc.astype(BF)


def _pack_chunk(k, flat, val):
    rows = lax.broadcasted_iota(jnp.int32, (C, T), 0) + k * C
    p_mat = (rows == flat).astype(BF)
    return jnp.dot(p_mat, val, preferred_element_type=jnp.float32).astype(BF)


def _unsort_add(j, flat, chunk, out_ref):
    hh = T // 2
    for half in range(2):
        fl = flat[half * hh:(half + 1) * hh][:, None]
        cols = lax.broadcasted_iota(jnp.int32, (hh, C), 1) + j * C
        g = (cols == fl).astype(BF)
        prod = jnp.dot(g, chunk, preferred_element_type=jnp.float32)
        sl = pl.ds(half * hh, hh)
        out_ref[sl, :] = out_ref[sl, :] + prod


def _body(val_ref, flat_ref, x_ref, shW_ref, eW_ref, out_ref,
          xsend_ref, recv_ref, yrecv_ref, ybuf_ref,
          ss1, rs1, ss2, rs2):
    me = lax.axis_index("i")

    barrier = pltpu.get_barrier_semaphore()
    for k in range(1, N):
        nbr = (me + k) % N
        pl.semaphore_signal(barrier, inc=1, device_id=(nbr,),
                            device_id_type=pl.DeviceIdType.MESH)
    pl.semaphore_wait(barrier, N - 1)

    val = val_ref[...]
    flat = flat_ref[0, :]

    send1 = []
    for k in range(1, N):
        dst = (me + k) % N
        xsend_ref[pl.ds(k * C, C), :] = _pack_chunk(k, flat, val)
        r = pltpu.make_async_remote_copy(
            src_ref=xsend_ref.at[pl.ds(k * C, C), :],
            dst_ref=recv_ref.at[pl.ds((N - k) * C, C), :],
            send_sem=ss1.at[k],
            recv_sem=rs1.at[N - k],
            device_id=(dst,),
            device_id_type=pl.DeviceIdType.MESH,
        )
        r.start()
        send1.append(r)

    out_ref[...] = jnp.dot(x_ref[...], shW_ref[...],
                           preferred_element_type=jnp.float32)

    xsend_ref[pl.ds(0, C), :] = _pack_chunk(0, flat, val)
    ybuf_ref[0] = _expert_chunk(xsend_ref[pl.ds(0, C), :], eW_ref)
    _unsort_add(0, flat, ybuf_ref[0], out_ref)

    send2 = []
    for j in range(1, N):
        s = (me + j) % N
        rwait = pltpu.make_async_remote_copy(
            src_ref=xsend_ref.at[pl.ds(0, C), :],
            dst_ref=recv_ref.at[pl.ds(j * C, C), :],
            send_sem=ss1.at[j],
            recv_sem=rs1.at[j],
            device_id=(s,),
            device_id_type=pl.DeviceIdType.MESH,
        )
        rwait.wait_recv()

        slot = (j - 1) % 2
        if len(send2) >= 2:
            send2[-2].wait_send()
        ybuf_ref[slot] = _expert_chunk(recv_ref[pl.ds(j * C, C), :], eW_ref)

        r = pltpu.make_async_remote_copy(
            src_ref=ybuf_ref.at[slot],
            dst_ref=yrecv_ref.at[pl.ds((N - j) * C, C), :],
            send_sem=ss2.at[j],
            recv_sem=rs2.at[N - j],
            device_id=(s,),
            device_id_type=pl.DeviceIdType.MESH,
        )
        r.start()
        send2.append(r)

    for jj in range(N - 1, 0, -1):
        rwait = pltpu.make_async_remote_copy(
            src_ref=ybuf_ref.at[0],
            dst_ref=yrecv_ref.at[pl.ds(jj * C, C), :],
            send_sem=ss2.at[jj],
            recv_sem=rs2.at[jj],
            device_id=((me + jj) % N,),
            device_id_type=pl.DeviceIdType.MESH,
        )
        rwait.wait_recv()
        _unsort_add(jj, flat, yrecv_ref[pl.ds(jj * C, C), :], out_ref)

    for r in send1:
        r.wait_send()
    for r in send2[-2:]:
        r.wait_send()


def kernel(x, router_W, route_idx, expert_W, shared_W):
    me = lax.axis_index("i")

    scores = jnp.dot(x, router_W, preferred_element_type=jnp.float32)
    scores = scores - scores.max(axis=1, keepdims=True)
    probs = jnp.exp(scores)
    probs = probs / probs.sum(axis=1, keepdims=True)

    e = route_idx[:, 0]
    n_exp = router_W.shape[1]
    oh_full = (e[:, None] == jnp.arange(n_exp, dtype=e.dtype)[None, :])
    p = jnp.sum(probs * oh_full.astype(probs.dtype), axis=1)

    dst = e // E_LOC
    eloc = e % E_LOC
    rel = (dst - me) % N

    oh_dst = (dst[:, None] == jnp.arange(N, dtype=dst.dtype)[None, :]).astype(jnp.int32)
    j_slot = jnp.sum(jnp.cumsum(oh_dst, axis=0) * oh_dst, axis=1) - 1
    flat = jnp.where(j_slot < C, rel * C + j_slot, N * C)

    xp = (x * p[:, None]).astype(BF)
    oh_e = (eloc[:, None] == jnp.arange(E_LOC, dtype=eloc.dtype)[None, :]).astype(BF)
    val = jnp.concatenate([xp, oh_e], axis=1)
    flat_i = flat.astype(jnp.int32)[None, :]

    out = pl.pallas_call(
        _body,
        out_shape=jax.ShapeDtypeStruct((T, H), jnp.float32),
        in_specs=[
            pl.BlockSpec(memory_space=pltpu.VMEM),
            pl.BlockSpec(memory_space=pltpu.VMEM),
            pl.BlockSpec(memory_space=pltpu.VMEM),
            pl.BlockSpec(memory_space=pltpu.VMEM),
            pl.BlockSpec(memory_space=pltpu.VMEM),
        ],
        out_specs=pl.BlockSpec(memory_space=pltpu.VMEM),
        scratch_shapes=[
            pltpu.VMEM((N * C, PAY), BF),
            pltpu.VMEM((N * C, PAY), BF),
            pltpu.VMEM((N * C, H), BF),
            pltpu.VMEM((2, C, H), BF),
            pltpu.SemaphoreType.DMA((N,)),
            pltpu.SemaphoreType.DMA((N,)),
            pltpu.SemaphoreType.DMA((N,)),
            pltpu.SemaphoreType.DMA((N,)),
        ],
        compiler_params=pltpu.CompilerParams(
            collective_id=0, vmem_limit_bytes=36 * 1024 * 1024),
    )(val, flat_i, x.astype(BF), shared_W.astype(BF), expert_W.astype(BF))

    return out


# device time: 167963 ns/iter; 1.4382x vs baseline; 1.1090x over previous
import jax
import jax.numpy as jnp
from jax import lax
from jax.experimental import pallas as pl
from jax.experimental.pallas import tpu as pltpu

N = 16
C = 192
D = 512
H = 1024
E_LOC = 8
PAY = 520
T = 2048

BF = jnp.bfloat16


def _expert_chunk(chunk, eW_ref):
    xp = chunk[:, :D]
    acc = None
    for j in range(E_LOC):
        contrib = jnp.dot(xp * chunk[:, D + j][:, None], eW_ref[j],
                          preferred_element_type=jnp.float32)
        acc = contrib if acc is None else acc + contrib
    return acc.astype(BF)


def _pack_chunk(k, flat, val):
    rows = lax.broadcasted_iota(jnp.int32, (C, T), 0) + k * C
    p_mat = (rows == flat).astype(BF)
    return jnp.dot(p_mat, val, preferred_element_type=jnp.float32).astype(BF)


def _unsort_add(j, flat, chunk, out_ref):
    hh = T // 2
    for half in range(2):
        fl = flat[half * hh:(half + 1) * hh][:, None]
        cols = lax.broadcasted_iota(jnp.int32, (hh, C), 1) + j * C
        g = (cols == fl).astype(BF)
        prod = jnp.dot(g, chunk, preferred_element_type=jnp.float32)
        sl = pl.ds(half * hh, hh)
        out_ref[sl, :] = out_ref[sl, :] + prod


def _body(val_ref, flat_ref, x_ref, shW_ref, eW_ref, out_ref,
          xsend_ref, recv_ref, yrecv_ref, ybuf_ref,
          ss1, rs1, ss2, rs2):
    me = lax.axis_index("i")

    barrier = pltpu.get_barrier_semaphore()
    for k in range(1, N):
        nbr = (me + k) % N
        pl.semaphore_signal(barrier, inc=1, device_id=(nbr,),
                            device_id_type=pl.DeviceIdType.MESH)
    pl.semaphore_wait(barrier, N - 1)

    val = val_ref[...]
    flat = flat_ref[0, :]

    send1 = []
    for k in range(1, N):
        dst = (me + k) % N
        xsend_ref[pl.ds(k * C, C), :] = _pack_chunk(k, flat, val)
        r = pltpu.make_async_remote_copy(
            src_ref=xsend_ref.at[pl.ds(k * C, C), :],
            dst_ref=recv_ref.at[pl.ds((N - k) * C, C), :],
            send_sem=ss1.at[k],
            recv_sem=rs1.at[N - k],
            device_id=(dst,),
            device_id_type=pl.DeviceIdType.MESH,
        )
        r.start()
        send1.append(r)

    out_ref[...] = jnp.dot(x_ref[...], shW_ref[...],
                           preferred_element_type=jnp.float32)

    xsend_ref[pl.ds(0, C), :] = _pack_chunk(0, flat, val)
    ybuf_ref[0] = _expert_chunk(xsend_ref[pl.ds(0, C), :], eW_ref)
    _unsort_add(0, flat, ybuf_ref[0], out_ref)

    send2 = []
    for j in range(N - 1, 0, -1):
        s = (me + j) % N
        rwait = pltpu.make_async_remote_copy(
            src_ref=xsend_ref.at[pl.ds(0, C), :],
            dst_ref=recv_ref.at[pl.ds(j * C, C), :],
            send_sem=ss1.at[j],
            recv_sem=rs1.at[j],
            device_id=(s,),
            device_id_type=pl.DeviceIdType.MESH,
        )
        rwait.wait_recv()

        slot = (j - 1) % 2
        if len(send2) >= 2:
            send2[-2].wait_send()
        ybuf_ref[slot] = _expert_chunk(recv_ref[pl.ds(j * C, C), :], eW_ref)

        r = pltpu.make_async_remote_copy(
            src_ref=ybuf_ref.at[slot],
            dst_ref=yrecv_ref.at[pl.ds((N - j) * C, C), :],
            send_sem=ss2.at[j],
            recv_sem=rs2.at[N - j],
            device_id=(s,),
            device_id_type=pl.DeviceIdType.MESH,
        )
        r.start()
        send2.append(r)

    for jj in range(1, N):
        rwait = pltpu.make_async_remote_copy(
            src_ref=ybuf_ref.at[0],
            dst_ref=yrecv_ref.at[pl.ds(jj * C, C), :],
            send_sem=ss2.at[jj],
            recv_sem=rs2.at[jj],
            device_id=((me + jj) % N,),
            device_id_type=pl.DeviceIdType.MESH,
        )
        rwait.wait_recv()
        _unsort_add(jj, flat, yrecv_ref[pl.ds(jj * C, C), :], out_ref)

    for r in send1:
        r.wait_send()
    for r in send2[-2:]:
        r.wait_send()


def kernel(x, router_W, route_idx, expert_W, shared_W):
    me = lax.axis_index("i")

    scores = jnp.dot(x, router_W, preferred_element_type=jnp.float32)
    scores = scores - scores.max(axis=1, keepdims=True)
    probs = jnp.exp(scores)
    probs = probs / probs.sum(axis=1, keepdims=True)

    e = route_idx[:, 0]
    n_exp = router_W.shape[1]
    oh_full = (e[:, None] == jnp.arange(n_exp, dtype=e.dtype)[None, :])
    p = jnp.sum(probs * oh_full.astype(probs.dtype), axis=1)

    dst = e // E_LOC
    eloc = e % E_LOC
    rel = (dst - me) % N

    oh_dst = (dst[:, None] == jnp.arange(N, dtype=dst.dtype)[None, :]).astype(jnp.int32)
    j_slot = jnp.sum(jnp.cumsum(oh_dst, axis=0) * oh_dst, axis=1) - 1
    flat = jnp.where(j_slot < C, rel * C + j_slot, N * C)

    xp = (x * p[:, None]).astype(BF)
    oh_e = (eloc[:, None] == jnp.arange(E_LOC, dtype=eloc.dtype)[None, :]).astype(BF)
    val = jnp.concatenate([xp, oh_e], axis=1)
    flat_i = flat.astype(jnp.int32)[None, :]

    out = pl.pallas_call(
        _body,
        out_shape=jax.ShapeDtypeStruct((T, H), jnp.float32),
        in_specs=[
            pl.BlockSpec(memory_space=pltpu.VMEM),
            pl.BlockSpec(memory_space=pltpu.VMEM),
            pl.BlockSpec(memory_space=pltpu.VMEM),
            pl.BlockSpec(memory_space=pltpu.VMEM),
            pl.BlockSpec(memory_space=pltpu.VMEM),
        ],
        out_specs=pl.BlockSpec(memory_space=pltpu.VMEM),
        scratch_shapes=[
            pltpu.VMEM((N * C, PAY), BF),
            pltpu.VMEM((N * C, PAY), BF),
            pltpu.VMEM((N * C, H), BF),
            pltpu.VMEM((2, C, H), BF),
            pltpu.SemaphoreType.DMA((N,)),
            pltpu.SemaphoreType.DMA((N,)),
            pltpu.SemaphoreType.DMA((N,)),
            pltpu.SemaphoreType.DMA((N,)),
        ],
        compiler_params=pltpu.CompilerParams(
            collective_id=0, vmem_limit_bytes=36 * 1024 * 1024),
    )(val, flat_i, x.astype(BF), shared_W.astype(BF), expert_W.astype(BF))

    return out


# device time: 162796 ns/iter; 1.4838x vs baseline; 1.0317x over previous
import jax
import jax.numpy as jnp
from jax import lax
from jax.experimental import pallas as pl
from jax.experimental.pallas import tpu as pltpu

N = 16
C = 192
D = 512
H = 1024
E_LOC = 8
PAY = 520
T = 2048

BF = jnp.bfloat16


def _expert_chunk(chunk, eW_ref):
    xp = chunk[:, :D]
    acc = None
    for j in range(E_LOC):
        contrib = jnp.dot(xp * chunk[:, D + j][:, None], eW_ref[j],
                          preferred_element_type=jnp.float32)
        acc = contrib if acc is None else acc + contrib
    return acc.astype(BF)


def _pack_chunk(k, flat, val):
    rows = lax.broadcasted_iota(jnp.int32, (C, T), 0) + k * C
    p_mat = (rows == flat).astype(BF)
    return jnp.dot(p_mat, val, preferred_element_type=jnp.float32).astype(BF)


def _unsort_add(j, flat, chunk, out_ref):
    hh = T // 2
    for half in range(2):
        fl = flat[half * hh:(half + 1) * hh][:, None]
        cols = lax.broadcasted_iota(jnp.int32, (hh, C), 1) + j * C
        g = (cols == fl).astype(BF)
        prod = jnp.dot(g, chunk, preferred_element_type=jnp.float32)
        sl = pl.ds(half * hh, hh)
        out_ref[sl, :] = out_ref[sl, :] + prod


def _body(val_ref, flat_ref, x_ref, shW_ref, eW_ref, out_ref,
          xsend_ref, recv_ref, yrecv_ref, ybuf_ref,
          ss1, rs1, ss2, rs2):
    me = lax.axis_index("i")

    barrier = pltpu.get_barrier_semaphore()
    for k in range(1, N):
        nbr = (me + k) % N
        pl.semaphore_signal(barrier, inc=1, device_id=(nbr,),
                            device_id_type=pl.DeviceIdType.MESH)
    pl.semaphore_wait(barrier, N - 1)

    val = val_ref[...]
    flat = flat_ref[0, :]

    send1 = []
    for k in range(1, N):
        dst = (me + k) % N
        xsend_ref[pl.ds(k * C, C), :] = _pack_chunk(k, flat, val)
        r = pltpu.make_async_remote_copy(
            src_ref=xsend_ref.at[pl.ds(k * C, C), :],
            dst_ref=recv_ref.at[pl.ds((N - k) * C, C), :],
            send_sem=ss1.at[k],
            recv_sem=rs1.at[N - k],
            device_id=(dst,),
            device_id_type=pl.DeviceIdType.MESH,
        )
        r.start()
        send1.append(r)

    out_ref[...] = jnp.dot(x_ref[...], shW_ref[...],
                           preferred_element_type=jnp.float32)

    xsend_ref[pl.ds(0, C), :] = _pack_chunk(0, flat, val)
    ybuf_ref[0] = _expert_chunk(xsend_ref[pl.ds(0, C), :], eW_ref)
    _unsort_add(0, flat, ybuf_ref[0], out_ref)

    send2 = []
    for j in range(N - 1, 0, -1):
        s = (me + j) % N
        rwait = pltpu.make_async_remote_copy(
            src_ref=xsend_ref.at[pl.ds(0, C), :],
            dst_ref=recv_ref.at[pl.ds(j * C, C), :],
            send_sem=ss1.at[j],
            recv_sem=rs1.at[j],
            device_id=(s,),
            device_id_type=pl.DeviceIdType.MESH,
        )
        rwait.wait_recv()

        slot = (j - 1) % 2
        if len(send2) >= 2:
            send2[-2].wait_send()
        ybuf_ref[slot] = _expert_chunk(recv_ref[pl.ds(j * C, C), :], eW_ref)

        r = pltpu.make_async_remote_copy(
            src_ref=ybuf_ref.at[slot],
            dst_ref=yrecv_ref.at[pl.ds((N - j) * C, C), :],
            send_sem=ss2.at[j],
            recv_sem=rs2.at[N - j],
            device_id=(s,),
            device_id_type=pl.DeviceIdType.MESH,
        )
        r.start()
        send2.append(r)

    for jj in range(1, N):
        rwait = pltpu.make_async_remote_copy(
            src_ref=ybuf_ref.at[0],
            dst_ref=yrecv_ref.at[pl.ds(jj * C, C), :],
            send_sem=ss2.at[jj],
            recv_sem=rs2.at[jj],
            device_id=((me + jj) % N,),
            device_id_type=pl.DeviceIdType.MESH,
        )
        rwait.wait_recv()
        _unsort_add(jj, flat, yrecv_ref[pl.ds(jj * C, C), :], out_ref)

    for r in send1:
        r.wait_send()
    for r in send2[-2:]:
        r.wait_send()


def kernel(x, router_W, route_idx, expert_W, shared_W):
    me = lax.axis_index("i")

    scores = jnp.dot(x, router_W, preferred_element_type=jnp.float32)
    scores = scores - scores.max(axis=1, keepdims=True)
    probs = jnp.exp(scores)
    probs = probs / probs.sum(axis=1, keepdims=True)

    e = route_idx[:, 0]
    n_exp = router_W.shape[1]
    oh_full = (e[:, None] == jnp.arange(n_exp, dtype=e.dtype)[None, :])
    p = jnp.sum(probs * oh_full.astype(probs.dtype), axis=1)

    dst = e // E_LOC
    eloc = e % E_LOC
    rel = (dst - me) % N

    oh_dst = (dst[:, None] == jnp.arange(N, dtype=dst.dtype)[None, :]).astype(jnp.float32)
    tri = (jnp.arange(T)[:, None] >= jnp.arange(T)[None, :]).astype(jnp.float32)
    csum = jnp.dot(tri, oh_dst, preferred_element_type=jnp.float32)
    j_slot = jnp.sum(csum * oh_dst, axis=1).astype(jnp.int32) - 1
    flat = jnp.where(j_slot < C, rel * C + j_slot, N * C)

    xp = (x * p[:, None]).astype(BF)
    oh_e = (eloc[:, None] == jnp.arange(E_LOC, dtype=eloc.dtype)[None, :]).astype(BF)
    val = jnp.concatenate([xp, oh_e], axis=1)
    flat_i = flat.astype(jnp.int32)[None, :]

    out = pl.pallas_call(
        _body,
        out_shape=jax.ShapeDtypeStruct((T, H), jnp.float32),
        in_specs=[
            pl.BlockSpec(memory_space=pltpu.VMEM),
            pl.BlockSpec(memory_space=pltpu.VMEM),
            pl.BlockSpec(memory_space=pltpu.VMEM),
            pl.BlockSpec(memory_space=pltpu.VMEM),
            pl.BlockSpec(memory_space=pltpu.VMEM),
        ],
        out_specs=pl.BlockSpec(memory_space=pltpu.VMEM),
        scratch_shapes=[
            pltpu.VMEM((N * C, PAY), BF),
            pltpu.VMEM((N * C, PAY), BF),
            pltpu.VMEM((N * C, H), BF),
            pltpu.VMEM((2, C, H), BF),
            pltpu.SemaphoreType.DMA((N,)),
            pltpu.SemaphoreType.DMA((N,)),
            pltpu.SemaphoreType.DMA((N,)),
            pltpu.SemaphoreType.DMA((N,)),
        ],
        compiler_params=pltpu.CompilerParams(
            collective_id=0, vmem_limit_bytes=36 * 1024 * 1024),
    )(val, flat_i, x.astype(BF), shared_W.astype(BF), expert_W.astype(BF))

    return out
